# baseline (speedup 1.0000x reference)
import functools

import jax
import jax.numpy as jnp
from jax import lax
from jax.experimental import pallas as pl
from jax.experimental.pallas import tpu as pltpu
from jax.experimental.pallas import tpu_sc as plsc

S = 7
NB = 2
NC = 80
LC = 5.0
LN = 0.5
BS = 128
CELLS = S * S
ROWS = BS * CELLS
NSLOT = float(BS * CELLS * NB)
NW = 8


def _tc1_body(pr_ref, bx_ref, lb_ref, tbl_ref, s0_ref, gb_ref):
    pr = pr_ref[...]
    bx = bx_ref[...]
    x1, y1, x2, y2 = bx[:, :, 0], bx[:, :, 1], bx[:, :, 2], bx[:, :, 3]
    cx = (x1 + x2) * 0.5
    cy = (y1 + y2) * 0.5
    sw = jnp.sqrt(jnp.maximum(x2 - x1, 1e-6))
    sh = jnp.sqrt(jnp.maximum(y2 - y1, 1e-6))
    gxf = cx * float(S)
    gyf = cy * float(S)
    gif = jnp.clip(jnp.floor(gxf), 0.0, float(S - 1))
    gjf = jnp.clip(jnp.floor(gyf), 0.0, float(S - 1))
    cellf = gjf * float(S) + gif
    labf = lb_ref[...].astype(jnp.float32)
    padgb = jnp.zeros((BS, 8), jnp.float32)
    gb_ref[...] = jnp.concatenate(
        [x1, y1, x2, y2, sw, sh, gif, gjf, gxf - gif, gyf - gjf, cellf,
         labf, padgb], axis=-1)
    logits = pr[..., NB * 5:]
    m = jnp.max(logits, axis=-1, keepdims=True)
    e = jnp.exp(logits - m)
    p = e / jnp.sum(e, axis=-1, keepdims=True)
    q = jnp.sum(p * p, axis=-1, keepdims=True)
    box = pr[..., :NB * 5]
    sig = 1.0 / (1.0 + jnp.exp(-box))
    col = lax.broadcasted_iota(jnp.int32, (BS, S, S, NB * 5), 3)
    is_raw = (col == 2) | (col == 3) | (col == 7) | (col == 8)
    mid = jnp.where(is_raw, box, sig)
    pad = jnp.zeros((BS, S, S, 37), jnp.float32)
    tbl = jnp.concatenate([p, mid, q, pad], axis=-1)
    tbl_ref[...] = tbl.reshape(ROWS, 128)
    s0 = jnp.sum(sig[..., 4] ** 2) + jnp.sum(sig[..., 9] ** 2)
    s0_ref[...] = jnp.broadcast_to(jnp.reshape(s0, (1, 1)), (1, 128))


def _iou16(ax1, ay1, ax2, ay2, bx1, by1, bx2, by2):
    ix1 = jnp.maximum(ax1, bx1)
    iy1 = jnp.maximum(ay1, by1)
    ix2 = jnp.minimum(ax2, bx2)
    iy2 = jnp.minimum(ay2, by2)
    inter = jnp.maximum(ix2 - ix1, 0.0) * jnp.maximum(iy2 - iy1, 0.0)
    aa = jnp.maximum(ax2 - ax1, 0.0) * jnp.maximum(ay2 - ay1, 0.0)
    ab = jnp.maximum(bx2 - bx1, 0.0) * jnp.maximum(by2 - by1, 0.0)
    return inter / (aa + ab - inter + 1e-9)


def _sc_body(tbl_hbm, gb_hbm, out_hbm,
             gb_v, idx0_v, idx1_v, rows0_v, rows1_v,
             outv, sem):
    w = lax.axis_index("s")

    @pl.when(w < NW)
    def _():
        b0 = w * 16
        pltpu.sync_copy(gb_hbm, gb_v)
        lane = lax.iota(jnp.int32, 16)
        bl = b0 + lane
        fc = lambda c: jnp.full((16,), c, jnp.int32)

        geom = []
        labs = []
        for i in range(10):
            gbg = lambda q: plsc.load_gather(gb_v, [bl, fc(q * 10 + i)])
            x1, y1, x2, y2 = gbg(0), gbg(1), gbg(2), gbg(3)
            sw, sh = gbg(4), gbg(5)
            gif, gjf = gbg(6), gbg(7)
            txh, tyh = gbg(8), gbg(9)
            cell = gbg(10).astype(jnp.int32)
            row = bl * CELLS + cell
            if i < 5:
                idx0_v[pl.ds(i * 16, 16)] = row
            else:
                idx1_v[pl.ds((i - 5) * 16, 16)] = row
            geom.append((x1, y1, x2, y2, sw, sh, gif, gjf, txh, tyh, cell))
            labs.append(gbg(11).astype(jnp.int32))

        cp0 = pltpu.async_copy(tbl_hbm.at[idx0_v], rows0_v, sem)
        cp1 = pltpu.async_copy(tbl_hbm.at[idx1_v], rows1_v, sem)
        cp0.wait()
        cp1.wait()

        slotkeys, clskeys, cellkeys = [], [], []
        a_l, c_l, q_l, p_l = [], [], [], []
        for i in range(10):
            x1, y1, x2, y2, sw, sh, gif, gjf, txh, tyh, cell = geom[i]
            rv = rows0_v if i < 5 else rows1_v
            r = (i % 5) * 16 + lane

            def gat(ci, _rv=rv, _r=r):
                return plsc.load_gather(_rv, [_r, ci])

            stx0, sty0 = gat(fc(80)), gat(fc(81))
            tw0, th0, sto0 = gat(fc(82)), gat(fc(83)), gat(fc(84))
            stx1, sty1 = gat(fc(85)), gat(fc(86))
            tw1, th1, sto1 = gat(fc(87)), gat(fc(88)), gat(fc(89))
            qv = gat(fc(90))
            pv = gat(labs[i])

            px0 = (stx0 + gif) / float(S)
            py0 = (sty0 + gjf) / float(S)
            pw0 = tw0 * tw0
            ph0 = th0 * th0
            iou0 = _iou16(px0 - 0.5 * pw0, py0 - 0.5 * ph0,
                          px0 + 0.5 * pw0, py0 + 0.5 * ph0, x1, y1, x2, y2)
            px1 = (stx1 + gif) / float(S)
            py1 = (sty1 + gjf) / float(S)
            pw1 = tw1 * tw1
            ph1 = th1 * th1
            iou1 = _iou16(px1 - 0.5 * pw1, py1 - 0.5 * ph1,
                          px1 + 0.5 * pw1, py1 + 0.5 * ph1, x1, y1, x2, y2)

            best = iou1 > iou0
            conf = jnp.where(best, iou1, iou0)
            stx = jnp.where(best, stx1, stx0)
            sty = jnp.where(best, sty1, sty0)
            tw = jnp.where(best, tw1, tw0)
            th = jnp.where(best, th1, th0)
            sto = jnp.where(best, sto1, sto0)

            slotkeys.append(cell * NB + best.astype(jnp.int32))
            clskeys.append(cell * NC + labs[i])
            cellkeys.append(cell)
            sq = lambda t: t * t
            a_l.append(sq(sto - conf)
                       + LC * (sq(stx - txh) + sq(sty - tyh)
                               + sq(tw - sw) + sq(th - sh)))
            c_l.append(sto * sto)
            q_l.append(qv)
            p_l.append(pv)

        ones = jnp.ones((16,), jnp.float32)
        zeros = jnp.zeros((16,), jnp.float32)

        def allmask(terms):
            if not terms:
                return ones
            m = terms[0]
            for t in terms[1:]:
                m = m & t
            return jnp.where(m, 1.0, 0.0)

        acc = [zeros] * 7
        for i in range(10):
            win = allmask([slotkeys[j] != slotkeys[i] for j in range(i + 1, 10)])
            fcls = allmask([clskeys[j] != clskeys[i] for j in range(i)])
            fcell = allmask([cellkeys[j] != cellkeys[i] for j in range(i)])
            acc[0] = acc[0] + win
            acc[1] = acc[1] + win * a_l[i]
            acc[2] = acc[2] + win * c_l[i]
            acc[3] = acc[3] + fcell
            acc[4] = acc[4] + fcell * q_l[i]
            acc[5] = acc[5] + fcls
            acc[6] = acc[6] + fcls * p_l[i]

        for k in range(7):
            outv[k, :] = acc[k]
        outv[7, :] = zeros
        pltpu.sync_copy(outv, out_hbm.at[w])


def _tc2_body(part_ref, s0_ref, out_ref):
    p = part_ref[...]
    rowq = lax.broadcasted_iota(jnp.int32, (NW, 8, 16), 1)

    def tot(qi):
        return jnp.sum(jnp.where(rowq == qi, p, 0.0))

    npos, a, c, ncell, qs, kcnt, ps = (tot(i) for i in range(7))
    s0 = s0_ref[0, 0]
    n_pos = jnp.maximum(npos, 1.0)
    n_neg = jnp.maximum(NSLOT - npos, 1.0)
    n_cell = jnp.maximum(ncell, 1.0)
    out_ref[...] = jnp.reshape(a / n_pos + LN * (s0 - c) / n_neg
                               + (qs + kcnt - 2.0 * ps) / n_cell, (1, 1))


def _make_sc():
    mesh = plsc.VectorSubcoreMesh(core_axis_name="c", subcore_axis_name="s",
                                  num_cores=1)
    return functools.partial(
        pl.kernel,
        mesh=mesh,
        compiler_params=pltpu.CompilerParams(needs_layout_passes=False),
        out_type=jax.ShapeDtypeStruct((NW, 8, 16), jnp.float32),
        scratch_types=[
            pltpu.VMEM((BS, 128), jnp.float32),
            pltpu.VMEM((80,), jnp.int32),
            pltpu.VMEM((80,), jnp.int32),
            pltpu.VMEM((80, 128), jnp.float32),
            pltpu.VMEM((80, 128), jnp.float32),
            pltpu.VMEM((8, 16), jnp.float32),
            pltpu.SemaphoreType.DMA,
        ],
    )(_sc_body)


def kernel(preds, boxes, labels):
    tbl, s0, gb = pl.pallas_call(
        _tc1_body,
        out_shape=(jax.ShapeDtypeStruct((ROWS, 128), jnp.float32),
                   jax.ShapeDtypeStruct((1, 128), jnp.float32),
                   jax.ShapeDtypeStruct((BS, 128), jnp.float32)),
    )(preds, boxes, labels)

    part = _make_sc()(tbl, gb)

    total = pl.pallas_call(
        _tc2_body,
        out_shape=jax.ShapeDtypeStruct((1, 1), jnp.float32),
    )(part, s0)
    return total.reshape(())


# speedup vs baseline: 1.0940x; 1.0940x over previous
import functools

import jax
import jax.numpy as jnp
from jax import lax
from jax.experimental import pallas as pl
from jax.experimental.pallas import tpu as pltpu
from jax.experimental.pallas import tpu_sc as plsc

S = 7
NB = 2
NC = 80
LC = 5.0
LN = 0.5
BS = 128
CELLS = S * S
ROWS = BS * CELLS
NSLOT = float(BS * CELLS * NB)
NW = 8


def _tc1_body(pr_ref, bx_ref, lb_ref, tbl_ref, s0_ref, gb_ref):
    pr = pr_ref[...]
    bx = bx_ref[...]
    x1, y1, x2, y2 = bx[:, :, 0], bx[:, :, 1], bx[:, :, 2], bx[:, :, 3]
    cx = (x1 + x2) * 0.5
    cy = (y1 + y2) * 0.5
    sw = jnp.sqrt(jnp.maximum(x2 - x1, 1e-6))
    sh = jnp.sqrt(jnp.maximum(y2 - y1, 1e-6))
    gxf = cx * float(S)
    gyf = cy * float(S)
    gif = jnp.clip(jnp.floor(gxf), 0.0, float(S - 1))
    gjf = jnp.clip(jnp.floor(gyf), 0.0, float(S - 1))
    cellf = gjf * float(S) + gif
    labf = lb_ref[...].astype(jnp.float32)
    padgb = jnp.zeros((BS, 8), jnp.float32)
    gb_ref[...] = jnp.concatenate(
        [x1, y1, x2, y2, sw, sh, gif, gjf, gxf - gif, gyf - gjf, cellf,
         labf, padgb], axis=-1)
    logits = pr[:, NB * 5:]
    m = jnp.max(logits, axis=-1, keepdims=True)
    e = jnp.exp(logits - m)
    p = e / jnp.sum(e, axis=-1, keepdims=True)
    q = jnp.sum(p * p, axis=-1, keepdims=True)
    box = pr[:, :NB * 5]
    sig = 1.0 / (1.0 + jnp.exp(-box))
    col = lax.broadcasted_iota(jnp.int32, (ROWS, NB * 5), 1)
    is_raw = (col == 2) | (col == 3) | (col == 7) | (col == 8)
    mid = jnp.where(is_raw, box, sig)
    pad = jnp.zeros((ROWS, 37), jnp.float32)
    tbl_ref[...] = jnp.concatenate([p, mid, q, pad], axis=-1)
    s0 = jnp.sum(sig[:, 4] ** 2) + jnp.sum(sig[:, 9] ** 2)
    s0_ref[...] = jnp.broadcast_to(jnp.reshape(s0, (1, 1)), (1, 128))


def _iou16(ax1, ay1, ax2, ay2, bx1, by1, bx2, by2):
    ix1 = jnp.maximum(ax1, bx1)
    iy1 = jnp.maximum(ay1, by1)
    ix2 = jnp.minimum(ax2, bx2)
    iy2 = jnp.minimum(ay2, by2)
    inter = jnp.maximum(ix2 - ix1, 0.0) * jnp.maximum(iy2 - iy1, 0.0)
    aa = jnp.maximum(ax2 - ax1, 0.0) * jnp.maximum(ay2 - ay1, 0.0)
    ab = jnp.maximum(bx2 - bx1, 0.0) * jnp.maximum(by2 - by1, 0.0)
    return inter / (aa + ab - inter + 1e-9)


def _sc_body(tbl_hbm, gb_hbm, out_hbm,
             gb_v, idx0_v, idx1_v, rows0_v, rows1_v,
             outv, sem):
    w = lax.axis_index("s")

    @pl.when(w < NW)
    def _():
        b0 = w * 16
        pltpu.sync_copy(gb_hbm, gb_v)
        lane = lax.iota(jnp.int32, 16)
        bl = b0 + lane
        fc = lambda c: jnp.full((16,), c, jnp.int32)

        geom = []
        labs = []
        for i in range(10):
            gbg = lambda q: plsc.load_gather(gb_v, [bl, fc(q * 10 + i)])
            x1, y1, x2, y2 = gbg(0), gbg(1), gbg(2), gbg(3)
            sw, sh = gbg(4), gbg(5)
            gif, gjf = gbg(6), gbg(7)
            txh, tyh = gbg(8), gbg(9)
            cell = gbg(10).astype(jnp.int32)
            row = bl * CELLS + cell
            if i < 5:
                idx0_v[pl.ds(i * 16, 16)] = row
            else:
                idx1_v[pl.ds((i - 5) * 16, 16)] = row
            geom.append((x1, y1, x2, y2, sw, sh, gif, gjf, txh, tyh, cell))
            labs.append(gbg(11).astype(jnp.int32))

        cp0 = pltpu.async_copy(tbl_hbm.at[idx0_v], rows0_v, sem)
        cp1 = pltpu.async_copy(tbl_hbm.at[idx1_v], rows1_v, sem)
        cp0.wait()
        cp1.wait()

        slotkeys, clskeys, cellkeys = [], [], []
        a_l, c_l, q_l, p_l = [], [], [], []
        for i in range(10):
            x1, y1, x2, y2, sw, sh, gif, gjf, txh, tyh, cell = geom[i]
            rv = rows0_v if i < 5 else rows1_v
            r = (i % 5) * 16 + lane

            def gat(ci, _rv=rv, _r=r):
                return plsc.load_gather(_rv, [_r, ci])

            stx0, sty0 = gat(fc(80)), gat(fc(81))
            tw0, th0, sto0 = gat(fc(82)), gat(fc(83)), gat(fc(84))
            stx1, sty1 = gat(fc(85)), gat(fc(86))
            tw1, th1, sto1 = gat(fc(87)), gat(fc(88)), gat(fc(89))
            qv = gat(fc(90))
            pv = gat(labs[i])

            px0 = (stx0 + gif) / float(S)
            py0 = (sty0 + gjf) / float(S)
            pw0 = tw0 * tw0
            ph0 = th0 * th0
            iou0 = _iou16(px0 - 0.5 * pw0, py0 - 0.5 * ph0,
                          px0 + 0.5 * pw0, py0 + 0.5 * ph0, x1, y1, x2, y2)
            px1 = (stx1 + gif) / float(S)
            py1 = (sty1 + gjf) / float(S)
            pw1 = tw1 * tw1
            ph1 = th1 * th1
            iou1 = _iou16(px1 - 0.5 * pw1, py1 - 0.5 * ph1,
                          px1 + 0.5 * pw1, py1 + 0.5 * ph1, x1, y1, x2, y2)

            best = iou1 > iou0
            conf = jnp.where(best, iou1, iou0)
            stx = jnp.where(best, stx1, stx0)
            sty = jnp.where(best, sty1, sty0)
            tw = jnp.where(best, tw1, tw0)
            th = jnp.where(best, th1, th0)
            sto = jnp.where(best, sto1, sto0)

            slotkeys.append(cell * NB + best.astype(jnp.int32))
            clskeys.append(cell * NC + labs[i])
            cellkeys.append(cell)
            sq = lambda t: t * t
            a_l.append(sq(sto - conf)
                       + LC * (sq(stx - txh) + sq(sty - tyh)
                               + sq(tw - sw) + sq(th - sh)))
            c_l.append(sto * sto)
            q_l.append(qv)
            p_l.append(pv)

        ones = jnp.ones((16,), jnp.float32)
        zeros = jnp.zeros((16,), jnp.float32)

        def allmask(terms):
            if not terms:
                return ones
            m = terms[0]
            for t in terms[1:]:
                m = m & t
            return jnp.where(m, 1.0, 0.0)

        acc = [zeros] * 7
        for i in range(10):
            win = allmask([slotkeys[j] != slotkeys[i] for j in range(i + 1, 10)])
            fcls = allmask([clskeys[j] != clskeys[i] for j in range(i)])
            fcell = allmask([cellkeys[j] != cellkeys[i] for j in range(i)])
            acc[0] = acc[0] + win
            acc[1] = acc[1] + win * a_l[i]
            acc[2] = acc[2] + win * c_l[i]
            acc[3] = acc[3] + fcell
            acc[4] = acc[4] + fcell * q_l[i]
            acc[5] = acc[5] + fcls
            acc[6] = acc[6] + fcls * p_l[i]

        for k in range(7):
            outv[k, :] = acc[k]
        outv[7, :] = zeros
        pltpu.sync_copy(outv, out_hbm.at[w])


def _tc2_body(part_ref, s0_ref, out_ref):
    p = part_ref[...]
    rowq = lax.broadcasted_iota(jnp.int32, (NW, 8, 16), 1)

    def tot(qi):
        return jnp.sum(jnp.where(rowq == qi, p, 0.0))

    npos, a, c, ncell, qs, kcnt, ps = (tot(i) for i in range(7))
    s0 = s0_ref[0, 0]
    n_pos = jnp.maximum(npos, 1.0)
    n_neg = jnp.maximum(NSLOT - npos, 1.0)
    n_cell = jnp.maximum(ncell, 1.0)
    out_ref[...] = jnp.reshape(a / n_pos + LN * (s0 - c) / n_neg
                               + (qs + kcnt - 2.0 * ps) / n_cell, (1, 1))


def _make_sc():
    mesh = plsc.VectorSubcoreMesh(core_axis_name="c", subcore_axis_name="s",
                                  num_cores=1)
    return functools.partial(
        pl.kernel,
        mesh=mesh,
        compiler_params=pltpu.CompilerParams(needs_layout_passes=False),
        out_type=jax.ShapeDtypeStruct((NW, 8, 16), jnp.float32),
        scratch_types=[
            pltpu.VMEM((BS, 128), jnp.float32),
            pltpu.VMEM((80,), jnp.int32),
            pltpu.VMEM((80,), jnp.int32),
            pltpu.VMEM((80, 128), jnp.float32),
            pltpu.VMEM((80, 128), jnp.float32),
            pltpu.VMEM((8, 16), jnp.float32),
            pltpu.SemaphoreType.DMA,
        ],
    )(_sc_body)


def kernel(preds, boxes, labels):
    tbl, s0, gb = pl.pallas_call(
        _tc1_body,
        out_shape=(jax.ShapeDtypeStruct((ROWS, 128), jnp.float32),
                   jax.ShapeDtypeStruct((1, 128), jnp.float32),
                   jax.ShapeDtypeStruct((BS, 128), jnp.float32)),
    )(preds.reshape(ROWS, 90), boxes, labels)

    part = _make_sc()(tbl, gb)

    total = pl.pallas_call(
        _tc2_body,
        out_shape=jax.ShapeDtypeStruct((1, 1), jnp.float32),
    )(part, s0)
    return total.reshape(())
